# Initial kernel scaffold; baseline (speedup 1.0000x reference)
import functools

import jax
import jax.numpy as jnp
from jax import lax
from jax.experimental import pallas as pl
from jax.experimental.pallas import tpu as pltpu
from jax.experimental.pallas import tpu_sc as plsc

NC = 2
NS = 16
NW = NC * NS
L = 16
BLK = 128


def _make_kernel(B, F, V, D):
    assert D == L
    assert B % (NW * BLK) == 0
    nblk = B // (NW * BLK)
    rows_per_blk = F * BLK

    mesh = plsc.VectorSubcoreMesh(core_axis_name="c", subcore_axis_name="s")

    @functools.partial(
        pl.kernel,
        out_type=jax.ShapeDtypeStruct((B,), jnp.float32),
        mesh=mesh,
        scratch_types=[
            pltpu.VMEM((2, F, BLK), jnp.int32),
            pltpu.VMEM((2, rows_per_blk, L), jnp.float32),
            pltpu.VMEM((F, L), jnp.float32),
            pltpu.VMEM((L * L,), jnp.float32),
            pltpu.VMEM((BLK,), jnp.float32),
            pltpu.SemaphoreType.DMA,
            pltpu.SemaphoreType.DMA,
        ],
    )
    def kern(tab_hbm, idx_hbm, w_hbm, out_hbm, idx_v, rows_v, w_v, colbuf,
             out_blk, sem0, sem1):
        sems = (sem0, sem1)
        wid = lax.axis_index("s") * NC + lax.axis_index("c")

        pltpu.sync_copy(w_hbm, w_v)
        wv = [w_v[f, :] for f in range(F)]
        iota16 = lax.iota(jnp.int32, L)
        colidx = iota16 * L

        def fire(slot, blk):
            g = wid * nblk + blk
            pltpu.sync_copy(idx_hbm.at[g], idx_v.at[slot])

            @pl.loop(0, F)
            def _(f):
                pltpu.async_copy(
                    tab_hbm.at[idx_v.at[slot, f]],
                    rows_v.at[slot, pl.ds(f * BLK, BLK)],
                    sems[slot],
                )

        def drain(slot):
            pltpu.make_async_copy(
                tab_hbm.at[pl.ds(0, rows_per_blk)], rows_v.at[slot], sems[slot]
            ).wait()

        def compute(slot, blk):
            @pl.loop(0, BLK // L)
            def _(grp):
                for l in range(L):
                    b = grp * L + l
                    acc = rows_v[slot, b, :] * wv[0]
                    for f in range(1, F):
                        acc = acc + rows_v[slot, f * BLK + b, :] * wv[f]
                    colbuf[pl.ds(l * L, L)] = acc
                tot = plsc.load_gather(colbuf, [colidx])
                for d in range(1, L):
                    tot = tot + plsc.load_gather(colbuf, [colidx + d])
                out_blk[pl.ds(grp * L, L)] = 1.0 / (1.0 + jnp.exp(-tot))

            base = wid * (nblk * BLK) + blk * BLK
            pltpu.sync_copy(out_blk, out_hbm.at[pl.ds(base, BLK)])

        fire(0, 0)
        for blk in range(nblk):
            slot = blk % 2
            if blk + 1 < nblk:
                fire(1 - slot, blk + 1)
            drain(slot)
            compute(slot, blk)

    return kern


def kernel(indices, tables, W):
    B, F = indices.shape
    _, V, D = tables.shape
    tab2 = tables.reshape(F * V, D)
    flat = indices.astype(jnp.int32) + (jnp.arange(F, dtype=jnp.int32) * V)[None, :]
    idx3 = flat.T.reshape(F, B // BLK, BLK).transpose(1, 0, 2)
    w2 = W.reshape(F, D).astype(jnp.float32)
    out = _make_kernel(B, F, V, D)(tab2, idx3, w2)
    return out.reshape(B, 1)


# speedup vs baseline: 8.3388x; 8.3388x over previous
import functools

import jax
import jax.numpy as jnp
from jax import lax
from jax.experimental import pallas as pl
from jax.experimental.pallas import tpu as pltpu
from jax.experimental.pallas import tpu_sc as plsc

NC = 2
NS = 16
NW = NC * NS
L = 16
BLK = 128


def _make_kernel(B, F, V, D):
    assert D == L
    assert B % (NW * BLK) == 0
    nblk = B // (NW * BLK)
    rows_per_blk = F * BLK

    mesh = plsc.VectorSubcoreMesh(core_axis_name="c", subcore_axis_name="s")

    @functools.partial(
        pl.kernel,
        out_type=jax.ShapeDtypeStruct((B,), jnp.float32),
        mesh=mesh,
        compiler_params=pltpu.CompilerParams(
            needs_layout_passes=False, use_tc_tiling_on_sc=False),
        scratch_types=[
            pltpu.VMEM((2, F, BLK), jnp.int32),
            pltpu.VMEM((2, rows_per_blk, L), jnp.float32),
            pltpu.VMEM((F, L), jnp.float32),
            pltpu.VMEM((L, L), jnp.float32),
            pltpu.VMEM((BLK,), jnp.float32),
            pltpu.SemaphoreType.DMA,
            pltpu.SemaphoreType.DMA,
        ],
    )
    def kern(tab_hbm, idx_hbm, w_hbm, out_hbm, idx_v, rows_v, w_v, colbuf,
             out_blk, sem0, sem1):
        sems = (sem0, sem1)
        wid = lax.axis_index("s") * NC + lax.axis_index("c")

        pltpu.sync_copy(w_hbm, w_v)
        wv = [w_v[f, :] for f in range(F)]
        iota16 = lax.iota(jnp.int32, L)

        def fire(slot, blk):
            g = wid * nblk + blk
            pltpu.sync_copy(idx_hbm.at[g], idx_v.at[slot])

            @pl.loop(0, F)
            def _(f):
                pltpu.async_copy(
                    tab_hbm.at[idx_v.at[slot, f]],
                    rows_v.at[slot, pl.ds(f * BLK, BLK)],
                    sems[slot],
                )

        def drain(slot):
            pltpu.make_async_copy(
                tab_hbm.at[pl.ds(0, rows_per_blk)], rows_v.at[slot], sems[slot]
            ).wait()

        def compute(slot, blk):
            @pl.loop(0, BLK // L)
            def _(grp):
                for l in range(L):
                    b = grp * L + l
                    acc = rows_v[slot, b, :] * wv[0]
                    for f in range(1, F):
                        acc = acc + rows_v[slot, f * BLK + b, :] * wv[f]
                    colbuf[l, :] = acc
                tot = plsc.load_gather(colbuf, [iota16, jnp.zeros((L,), jnp.int32)])
                for d in range(1, L):
                    tot = tot + plsc.load_gather(
                        colbuf, [iota16, jnp.full((L,), d, jnp.int32)])
                out_blk[pl.ds(grp * L, L)] = 1.0 / (1.0 + jnp.exp(-tot))

            base = wid * (nblk * BLK) + blk * BLK
            pltpu.sync_copy(out_blk, out_hbm.at[pl.ds(base, BLK)])

        fire(0, 0)
        for blk in range(nblk):
            slot = blk % 2
            if blk + 1 < nblk:
                fire(1 - slot, blk + 1)
            drain(slot)
            compute(slot, blk)

    return kern


def kernel(indices, tables, W):
    B, F = indices.shape
    _, V, D = tables.shape
    tab2 = tables.reshape(F * V, D)
    flat = indices.astype(jnp.int32) + (jnp.arange(F, dtype=jnp.int32) * V)[None, :]
    idx3 = flat.T.reshape(F, B // BLK, BLK).transpose(1, 0, 2)
    w2 = W.reshape(F, D).astype(jnp.float32)
    out = _make_kernel(B, F, V, D)(tab2, idx3, w2)
    return out.reshape(B, 1)
